# Initial kernel scaffold; baseline (speedup 1.0000x reference)
import jax
import jax.numpy as jnp
from jax.experimental import pallas as pl


def kernel(feats, soft_mask, conv_w_f, fc1_w_f, fc1_b_f, fc2_w_f, fc2_b_f, conv_w_b, fc1_w_b, fc1_b_b, fc2_w_b, fc2_b_b):
    raise NotImplementedError("write your pallas kernel here")


# speedup vs baseline: 1.9874x; 1.9874x over previous
import functools

import jax
import jax.numpy as jnp
from jax.experimental import pallas as pl

K = 100
C = 1024
HW = 1024
KP = 128
KH = 50
KHP = 64


def _branch(feats, vcol, vrow, m_row, relu_cwf_row, colnorm2_row,
            fc1w, fc1b, fc2w, fc2b):
    ii = jax.lax.broadcasted_iota(jnp.int32, (HW, HW), 1)
    jj = jax.lax.broadcasted_iota(jnp.int32, (HW, HW), 0)
    gt = vrow > vcol
    tie = (vrow == vcol) & (ii < jj)
    mt = jnp.where(gt | tie, 1, 0)
    rank_col = jnp.sum(mt, axis=1, keepdims=True)
    kio = jax.lax.broadcasted_iota(jnp.int32, (HW, KP), 1)
    pt = jnp.where((rank_col == kio) & (kio < K), 1.0, 0.0)

    x_col = jax.lax.dot_general(pt, relu_cwf_row, (((0,), (1,)), ((), ())),
                                preferred_element_type=jnp.float32)
    h = jax.lax.dot_general(fc1w, x_col, (((1,), (0,)), ((), ())),
                            preferred_element_type=jnp.float32) + fc1b
    y = jax.lax.dot_general(fc2w, h, (((1,), (0,)), ((), ())),
                            preferred_element_type=jnp.float32) + fc2b
    gate_col = jax.nn.sigmoid(y)

    pn2_col = jax.lax.dot_general(pt, colnorm2_row, (((0,), (1,)), ((), ())),
                                  preferred_element_type=jnp.float32)
    pn_col = jnp.sqrt(gate_col * gate_col * pn2_col + 1e-12)
    on_row = jnp.sqrt(colnorm2_row * m_row + 1e-12)

    protos = jax.lax.dot_general(feats, pt, (((1,), (0,)), ((), ())),
                                 preferred_element_type=jnp.float32)
    raw = jax.lax.dot_general(protos, feats, (((0,), (0,)), ((), ())),
                              preferred_element_type=jnp.float32)
    att = (raw * m_row) * gate_col / (pn_col * on_row + 1e-8)
    att = jnp.maximum(att, 0.0)
    return jnp.max(att, axis=0, keepdims=True)


def _body(feats_ref, soft_ref, soft_t_ref,
          cwf_f_ref, fc1w_f_ref, fc1b_f_ref, fc2w_f_ref, fc2b_f_ref,
          cwf_b_ref, fc1w_b_ref, fc1b_b_ref, fc2w_b_ref, fc2b_b_ref,
          out_ref):
    feats = feats_ref[0]
    soft = soft_ref[0]
    soft_t = soft_t_ref[0]
    s0r = soft[0:1, :]
    s1r = soft[1:2, :]
    fg_row = jnp.where(s1r > s0r, 1.0, 0.0)
    bg_row = 1.0 - fg_row

    colnorm2_row = jnp.sum(feats * feats, axis=0, keepdims=True)
    cwf_f = jax.lax.dot_general(cwf_f_ref[...], feats, (((1,), (0,)), ((), ())),
                                preferred_element_type=jnp.float32)
    cwf_b = jax.lax.dot_general(cwf_b_ref[...], feats, (((1,), (0,)), ((), ())),
                                preferred_element_type=jnp.float32)

    fore = _branch(feats, soft_t[:, 1:2], s1r, bg_row,
                   jnp.maximum(cwf_f, 0.0), colnorm2_row,
                   fc1w_f_ref[...], fc1b_f_ref[...], fc2w_f_ref[...], fc2b_f_ref[...])
    back = _branch(feats, soft_t[:, 0:1], s0r, fg_row,
                   jnp.maximum(cwf_b, 0.0), colnorm2_row,
                   fc1w_b_ref[...], fc1b_b_ref[...], fc2w_b_ref[...], fc2b_b_ref[...])

    out_ref[0] = feats * (1.0 + s1r - back + fore)


def _pad2(a, r, c):
    out = jnp.zeros((r, c), a.dtype)
    return out.at[:a.shape[0], :a.shape[1]].set(a)


def kernel(feats, soft_mask, conv_w_f, fc1_w_f, fc1_b_f, fc2_w_f, fc2_b_f,
           conv_w_b, fc1_w_b, fc1_b_b, fc2_w_b, fc2_b_b):
    b, c, h, w = feats.shape
    hw = h * w
    feats3 = feats.reshape(b, c, hw)
    soft3 = soft_mask.reshape(b, 2, hw)
    soft3_t = jnp.transpose(soft3, (0, 2, 1))

    args = (
        feats3, soft3, soft3_t,
        conv_w_f.reshape(1, c),
        _pad2(fc1_w_f, KHP, KP), _pad2(fc1_b_f.reshape(KH, 1), KHP, 1),
        _pad2(fc2_w_f, KP, KHP), _pad2(fc2_b_f.reshape(K, 1), KP, 1),
        conv_w_b.reshape(1, c),
        _pad2(fc1_w_b, KHP, KP), _pad2(fc1_b_b.reshape(KH, 1), KHP, 1),
        _pad2(fc2_w_b, KP, KHP), _pad2(fc2_b_b.reshape(K, 1), KP, 1),
    )

    def fixed(shape):
        return pl.BlockSpec(shape, lambda i: (0,) * len(shape))

    out3 = pl.pallas_call(
        _body,
        grid=(b,),
        in_specs=[
            pl.BlockSpec((1, c, hw), lambda i: (i, 0, 0)),
            pl.BlockSpec((1, 2, hw), lambda i: (i, 0, 0)),
            pl.BlockSpec((1, hw, 2), lambda i: (i, 0, 0)),
            fixed((1, c)),
            fixed((KHP, KP)), fixed((KHP, 1)),
            fixed((KP, KHP)), fixed((KP, 1)),
            fixed((1, c)),
            fixed((KHP, KP)), fixed((KHP, 1)),
            fixed((KP, KHP)), fixed((KP, 1)),
        ],
        out_specs=pl.BlockSpec((1, c, hw), lambda i: (i, 0, 0)),
        out_shape=jax.ShapeDtypeStruct((b, c, hw), jnp.float32),
    )(*args)
    return out3.reshape(b, c, h, w)
